# baseline (speedup 1.0000x reference)
import functools

import jax
import jax.numpy as jnp
from jax import lax
from jax.experimental import pallas as pl
from jax.experimental.pallas import tpu as pltpu
from jax.experimental.pallas import tpu_sc as plsc

BATCH = 16384
DIM = 16
NUM_WORKERS = 32
BPW = BATCH // NUM_WORKERS
GROUPS = BPW // 16


def _mf_body(user_h, pos_h, neg_h, ue_h, ie_h, pos_out_h, neg_out_h,
             uidx_v, pidx_v, nidx_v, urows_v, prows_v, nrows_v,
             tp_v, tn_v, pscore_v, nscore_v, sem):
    c = lax.axis_index("c")
    s = lax.axis_index("s")
    wid = s * 2 + c
    base = wid * BPW

    pltpu.sync_copy(user_h.at[pl.ds(base, BPW)], uidx_v)
    pltpu.sync_copy(pos_h.at[pl.ds(base, BPW)], pidx_v)
    pltpu.sync_copy(neg_h.at[pl.ds(base, BPW)], nidx_v)

    cp_u = pltpu.async_copy(ue_h.at[uidx_v], urows_v, sem)
    cp_p = pltpu.async_copy(ie_h.at[pidx_v], prows_v, sem)
    cp_n = pltpu.async_copy(ie_h.at[nidx_v], nrows_v, sem)
    cp_u.wait()
    cp_p.wait()
    cp_n.wait()

    lane = lax.iota(jnp.int32, 16)

    def group(g, carry):
        def row(j, carry2):
            u = urows_v[g * 16 + j, :]
            p = prows_v[g * 16 + j, :]
            n = nrows_v[g * 16 + j, :]
            col_idx = lane * 16 + j
            plsc.store_scatter(tp_v, [col_idx], u * p)
            plsc.store_scatter(tn_v, [col_idx], u * n)
            return carry2

        lax.fori_loop(0, 16, row, 0)
        accp = jnp.zeros((16,), jnp.float32)
        accn = jnp.zeros((16,), jnp.float32)
        for d in range(DIM):
            accp = accp + tp_v[pl.ds(d * 16, 16)]
            accn = accn + tn_v[pl.ds(d * 16, 16)]
        pscore_v[pl.ds(g * 16, 16)] = accp
        nscore_v[pl.ds(g * 16, 16)] = accn
        return carry

    lax.fori_loop(0, GROUPS, group, 0)

    pltpu.sync_copy(pscore_v, pos_out_h.at[pl.ds(base, BPW)])
    pltpu.sync_copy(nscore_v, neg_out_h.at[pl.ds(base, BPW)])


@jax.jit
def _mf(user, pos, neg, user_embedding, item_embedding):
    mesh = plsc.VectorSubcoreMesh(core_axis_name="c", subcore_axis_name="s")
    f = functools.partial(
        pl.kernel,
        out_type=(
            jax.ShapeDtypeStruct((BATCH,), jnp.float32),
            jax.ShapeDtypeStruct((BATCH,), jnp.float32),
        ),
        mesh=mesh,
        scratch_types=[
            pltpu.VMEM((BPW,), jnp.int32),
            pltpu.VMEM((BPW,), jnp.int32),
            pltpu.VMEM((BPW,), jnp.int32),
            pltpu.VMEM((BPW, DIM), jnp.float32),
            pltpu.VMEM((BPW, DIM), jnp.float32),
            pltpu.VMEM((BPW, DIM), jnp.float32),
            pltpu.VMEM((16 * DIM,), jnp.float32),
            pltpu.VMEM((16 * DIM,), jnp.float32),
            pltpu.VMEM((BPW,), jnp.float32),
            pltpu.VMEM((BPW,), jnp.float32),
            pltpu.SemaphoreType.DMA,
        ],
        compiler_params=pltpu.CompilerParams(
            needs_layout_passes=False, use_tc_tiling_on_sc=False),
    )(_mf_body)
    return f(user, pos, neg, user_embedding, item_embedding)


def kernel(user, pos, neg, user_embedding, item_embedding):
    return _mf(user, pos, neg, user_embedding, item_embedding)
